# baseline (speedup 1.0000x reference)
import jax
import jax.numpy as jnp
from jax import lax
from jax.experimental import pallas as pl
from jax.experimental.pallas import tpu as pltpu

_LAMB = max(5.0, 1500.0 / 1.001)
_DENOM = 1.0 + _LAMB
_B = 4096
_C = 1000
_BR = 512
_NS = _B // _BR
_NBUF = 3


def _body(cos_hbm, phi_hbm, tgt_ref, iota_ref, out_ref, *scr):
    cbufs = scr[:_NBUF]
    pbufs = scr[_NBUF:2 * _NBUF]
    csems = scr[2 * _NBUF:3 * _NBUF]
    psems = scr[3 * _NBUF:4 * _NBUF]

    def cp(k, start):
        b = k % _NBUF
        cc = pltpu.make_async_copy(
            cos_hbm.at[pl.ds(k * _BR, _BR), :], cbufs[b], csems[b])
        pc = pltpu.make_async_copy(
            phi_hbm.at[pl.ds(k * _BR, _BR), :], pbufs[b], psems[b])
        if start:
            cc.start()
            pc.start()
        else:
            cc.wait()
            pc.wait()

    for k in range(_NBUF):
        cp(k, True)

    ones = jnp.ones((_C, 1), jnp.float32)
    acc = jnp.zeros((1, 1), jnp.float32)
    for k in range(_NS):
        b = k % _NBUF
        cp(k, False)
        cosb = cbufs[b][...]
        phib = pbufs[b][...]
        tgt = tgt_ref[pl.ds(k * _BR, _BR), :]
        mask = iota_ref[...] == tgt
        m0 = jnp.max(cosb, axis=1, keepdims=True)
        e = jnp.exp(cosb - m0)
        s0 = lax.dot_general(e, ones, (((1,), (0,)), ((), ())),
                             preferred_element_type=jnp.float32)
        ct = lax.dot_general(jnp.where(mask, cosb, 0.0), ones,
                             (((1,), (0,)), ((), ())),
                             preferred_element_type=jnp.float32)
        pt_ = lax.dot_general(jnp.where(mask, phib, 0.0), ones,
                              (((1,), (0,)), ((), ())),
                              preferred_element_type=jnp.float32)
        if k + _NBUF < _NS:
            cp(k + _NBUF, True)
        mt = ct + (pt_ - ct) / _DENOM
        m = jnp.maximum(m0, mt)
        s = s0 * jnp.exp(m0 - m) - jnp.exp(ct - m) + jnp.exp(mt - m)
        logpt = mt - m - jnp.log(s)
        pt = jnp.exp(logpt)
        omp = 1.0 - pt
        acc += jnp.sum(omp * omp * logpt, keepdims=True)
    out_ref[...] = -acc / _B


def kernel(cos_theta, phi_theta, xlen, target):
    del xlen
    tgt_col = target.reshape(_B, 1)
    iota_row = jnp.arange(_C, dtype=jnp.int32).reshape(1, _C)
    r = pl.pallas_call(
        _body,
        in_specs=[
            pl.BlockSpec(memory_space=pl.ANY),
            pl.BlockSpec(memory_space=pl.ANY),
            pl.BlockSpec(memory_space=pltpu.VMEM),
            pl.BlockSpec(memory_space=pltpu.VMEM),
        ],
        out_specs=pl.BlockSpec(memory_space=pltpu.VMEM),
        out_shape=jax.ShapeDtypeStruct((1, 1), jnp.float32),
        scratch_shapes=(
            [pltpu.VMEM((_BR, _C), jnp.float32) for _ in range(2 * _NBUF)]
            + [pltpu.SemaphoreType.DMA for _ in range(2 * _NBUF)]),
        compiler_params=pltpu.CompilerParams(
            vmem_limit_bytes=100 * 1024 * 1024),
    )(cos_theta, phi_theta, tgt_col, iota_row)
    return r[0, 0]
